# baseline (speedup 1.0000x reference)
import jax
import jax.numpy as jnp
from jax.experimental import pallas as pl
from jax.experimental.pallas import tpu as pltpu

_N = 100000
_B = 12800
_NB = -(-_N // _B)


def _tc_body(vt_ref, m_ref, c_ref, jt_ref, acc_ref, q_ref):
    i = pl.program_id(0)

    @pl.when(i == 0)
    def _():
        acc_ref[...] = jnp.zeros_like(acc_ref)

    @pl.when(i < _NB)
    def _():
        lane = jax.lax.broadcasted_iota(jnp.int32, (1, _B), 1)
        mask = (i * _B + lane) < _N
        prod = vt_ref[...] * m_ref[...]
        part = jnp.sum(jnp.where(mask, prod, 0.0), axis=1, keepdims=True)
        acc_ref[...] += part

    @pl.when(i == _NB - 1)
    def _():
        c = acc_ref[...]
        q_ref[...] = jnp.concatenate(
            [c, jnp.zeros((3, 1), jnp.float32)], axis=0)
        c_ref[...] = c

    @pl.when(i >= _NB)
    def _():
        jt_ref[...] = q_ref[...] * m_ref[...]


def _tc_call(vt, m2):
    c, jt = pl.pallas_call(
        _tc_body,
        grid=(2 * _NB,),
        in_specs=[
            pl.BlockSpec((3, _B), lambda i: (0, jnp.where(i < _NB, i, 0))),
            pl.BlockSpec((1, _B), lambda i: (0, jnp.where(i < _NB, i, i - _NB))),
        ],
        out_specs=[
            pl.BlockSpec((3, 1), lambda i: (0, 0)),
            pl.BlockSpec((6, _B), lambda i: (0, jnp.where(i < _NB, 0, i - _NB))),
        ],
        out_shape=[
            jax.ShapeDtypeStruct((3, 1), jnp.float32),
            jax.ShapeDtypeStruct((6, _N), jnp.float32),
        ],
        scratch_shapes=[
            pltpu.VMEM((3, 1), jnp.float32),
            pltpu.VMEM((6, 1), jnp.float32),
        ],
        compiler_params=pltpu.CompilerParams(
            dimension_semantics=("arbitrary",)),
    )(vt, m2)
    return c, jt


def kernel(r, v, batch, z, m):
    c, jt = _tc_call(v.T, m[None, :])
    return (c, jt.T)


# speedup vs baseline: 2.0403x; 2.0403x over previous
import jax
import jax.numpy as jnp
from jax.experimental import pallas as pl
from jax.experimental.pallas import tpu as pltpu

_N = 100000


def _tc_body(vt_ref, m_ref, c_ref, jt_ref):
    m = m_ref[...]
    prod = vt_ref[...] * m
    c = jnp.sum(prod, axis=1, keepdims=True)
    c_ref[...] = jnp.concatenate(
        [c[0:1, :], c[1:2, :], c[2:3, :]], axis=1)
    q = jnp.concatenate(
        [c, jnp.zeros((3, 1), jnp.float32)], axis=0)
    jt_ref[...] = q * m


def _tc_call(vt, m2):
    return pl.pallas_call(
        _tc_body,
        out_shape=[
            jax.ShapeDtypeStruct((1, 3), jnp.float32),
            jax.ShapeDtypeStruct((6, _N), jnp.float32),
        ],
    )(vt, m2)


def kernel(r, v, batch, z, m):
    c, jt = _tc_call(v.T, m[None, :])
    return (c.reshape(3, 1), jt.T)


# speedup vs baseline: 2.9145x; 1.4285x over previous
import jax
import jax.numpy as jnp
from jax.experimental import pallas as pl
from jax.experimental.pallas import tpu as pltpu

_N = 100000


def _tc_body(vt_ref, m_ref, c_ref, jt_ref):
    m = m_ref[...].reshape(1, _N)
    prod = vt_ref[...] * m
    c = jnp.sum(prod, axis=1, keepdims=True)
    c_ref[...] = jnp.concatenate(
        [c[0:1, :], c[1:2, :], c[2:3, :]], axis=1)
    q = jnp.concatenate(
        [c, jnp.zeros((3, 1), jnp.float32)], axis=0)
    jt_ref[...] = q * m


def _tc_call(vt, m2):
    return pl.pallas_call(
        _tc_body,
        out_shape=[
            jax.ShapeDtypeStruct((1, 3), jnp.float32),
            jax.ShapeDtypeStruct((6, _N), jnp.float32),
        ],
    )(vt, m2)


def kernel(r, v, batch, z, m):
    c, jt = _tc_call(v.T, m)
    return (c.reshape(3, 1), jt.T)
